# baseline (speedup 1.0000x reference)
import jax
import jax.numpy as jnp
from jax import lax
from jax.experimental import pallas as pl
from jax.experimental.pallas import tpu as pltpu
from jax.experimental.pallas import tpu_sc as plsc

_BATCH = 16384
_DIM = 64
_NTAB = 4
_NC = 2
_NS = 16
_NW = _NC * _NS
_BPW = _BATCH // _NW
_CHUNK = 128
_NCHUNK = _BPW // _CHUNK
_NBUF = 4
_LAG = 2

_MESH = dict(
    mesh=plsc.VectorSubcoreMesh(core_axis_name="c", subcore_axis_name="s"),
    compiler_params=pltpu.CompilerParams(use_tc_tiling_on_sc=False),
)


def _gather_body(idx_h, table_h, out_h, idx_v, bufs, gsems, ssems):
    wid = lax.axis_index("s") * _NC + lax.axis_index("c")
    base = wid * _BPW
    pltpu.sync_copy(idx_h.at[pl.ds(base, _BPW)], idx_v)

    hg = [None] * _NCHUNK
    hs = [None] * _NCHUNK

    def fire_store(j):
        k = j % _NBUF
        hg[j].wait()
        hs[j] = pltpu.async_copy(
            bufs[k], out_h.at[pl.ds(base + j * _CHUNK, _CHUNK), :], ssems[k])

    for j in range(_NCHUNK):
        k = j % _NBUF
        if j >= _NBUF:
            hs[j - _NBUF].wait()
        hg[j] = pltpu.async_copy(
            table_h.at[idx_v.at[pl.ds(j * _CHUNK, _CHUNK)]], bufs[k],
            gsems[k])
        if j >= _LAG:
            fire_store(j - _LAG)
    for j in range(max(0, _NCHUNK - _LAG), _NCHUNK):
        fire_store(j)
    for j in range(max(0, _NCHUNK - _NBUF), _NCHUNK):
        hs[j].wait()


def _gather_call(idx, table):
    def body(ih, th, oh, idx_v, b0, b1, b2, b3, g0, g1, g2, g3,
             s0, s1, s2, s3):
        _gather_body(ih, th, oh, idx_v, (b0, b1, b2, b3),
                     (g0, g1, g2, g3), (s0, s1, s2, s3))

    f = pl.kernel(
        body,
        out_type=jax.ShapeDtypeStruct((_BATCH, _DIM), jnp.float32),
        scratch_types=[
            pltpu.VMEM((_BPW,), jnp.int32),
        ] + [pltpu.VMEM((_CHUNK, _DIM), jnp.float32)] * _NBUF
          + [pltpu.SemaphoreType.DMA] * (2 * _NBUF),
        **_MESH,
    )
    return f(idx, table)


def _assemble_call(o0, o1, o2, o3):
    def body(a0, a1, a2, a3, out_h, *scr):
        ins = (a0, a1, a2, a3)
        bufs = scr[:2 * _NTAB]
        gsems = scr[2 * _NTAB:2 * _NTAB + 2]
        ssems = scr[2 * _NTAB + 2:]
        wid = lax.axis_index("s") * _NC + lax.axis_index("c")
        base = wid * _BPW
        hg = [[None] * _NTAB for _ in range(2)]
        hs = [[None] * _NTAB for _ in range(2)]

        def fire_loads(j, pb):
            for c in range(_NTAB):
                hg[pb][c] = pltpu.async_copy(
                    ins[c].at[pl.ds(base + j * _CHUNK, _CHUNK), :],
                    bufs[pb * _NTAB + c], gsems[pb])

        def fire_stores(j, pb):
            for c in range(_NTAB):
                hg[pb][c].wait()
                if hs[pb][c] is not None:
                    hs[pb][c].wait()
                hs[pb][c] = pltpu.async_copy(
                    bufs[pb * _NTAB + c],
                    out_h.at[pl.ds(base + j * _CHUNK, _CHUNK),
                             pl.ds(c * _DIM, _DIM)],
                    ssems[pb])

        fire_loads(0, 0)
        for j in range(_NCHUNK):
            pb = j % 2
            if j + 1 < _NCHUNK:
                fire_loads(j + 1, 1 - pb)
            fire_stores(j, pb)
        for pb in range(2):
            for c in range(_NTAB):
                if hs[pb][c] is not None:
                    hs[pb][c].wait()

    f = pl.kernel(
        body,
        out_type=jax.ShapeDtypeStruct((_BATCH, _NTAB * _DIM), jnp.float32),
        scratch_types=(
            [pltpu.VMEM((_CHUNK, _DIM), jnp.float32)] * (2 * _NTAB)
            + [pltpu.SemaphoreType.DMA] * 4
        ),
        **_MESH,
    )
    return f(o0, o1, o2, o3)


def kernel(uid, rid, ing, nut, user_table, recipe_table, ingredient_table,
           nutrition_table):
    idxs = [x.astype(jnp.int32) for x in (uid, rid, ing, nut)]
    tabs = [t.astype(jnp.float32) for t in
            (user_table, recipe_table, ingredient_table, nutrition_table)]
    outs = [_gather_call(i, t) for i, t in zip(idxs, tabs)]
    return _assemble_call(*outs)


# speedup vs baseline: 1.0427x; 1.0427x over previous
import jax
import jax.numpy as jnp
from jax import lax
from jax.experimental import pallas as pl
from jax.experimental.pallas import tpu as pltpu
from jax.experimental.pallas import tpu_sc as plsc

_BATCH = 16384
_DIM = 64
_NTAB = 4
_NC = 2
_NS = 16
_NW = _NC * _NS
_BPW = _BATCH // _NW
_CHUNK = 128
_NCHUNK = _BPW // _CHUNK
_NCHUNKS_TOTAL = _NTAB * _NCHUNK
_NBUF = 4
_LAG = 2


def _body(uid_h, rid_h, ing_h, nut_h, user_t, recipe_t, ingredient_t,
          nutrition_t, out_hbm, idx_v, bufs, gsems, ssems):
    tables = (user_t, recipe_t, ingredient_t, nutrition_t)
    idx_hbms = (uid_h, rid_h, ing_h, nut_h)
    wid = lax.axis_index("s") * _NC + lax.axis_index("c")
    base = wid * _BPW

    for c in range(_NTAB):
        pltpu.sync_copy(idx_hbms[c].at[pl.ds(base, _BPW)], idx_v.at[c])

    chunks = [(c, j) for c in range(_NTAB) for j in range(_NCHUNK)]
    hg = [None] * _NCHUNKS_TOTAL
    hs = [None] * _NCHUNKS_TOTAL

    def fire_store(i):
        c, j = chunks[i]
        k = i % _NBUF
        hg[i].wait()
        hs[i] = pltpu.async_copy(
            bufs[k],
            out_hbm.at[pl.ds(base + j * _CHUNK, _CHUNK),
                       pl.ds(c * _DIM, _DIM)],
            ssems[k])

    for i, (c, j) in enumerate(chunks):
        k = i % _NBUF
        if i >= _NBUF:
            hs[i - _NBUF].wait()
        hg[i] = pltpu.async_copy(
            tables[c].at[idx_v.at[c, pl.ds(j * _CHUNK, _CHUNK)]],
            bufs[k], gsems[k])
        if i >= _LAG:
            fire_store(i - _LAG)
    for i in range(_NCHUNKS_TOTAL - _LAG, _NCHUNKS_TOTAL):
        fire_store(i)
    for i in range(_NCHUNKS_TOTAL - _NBUF, _NCHUNKS_TOTAL):
        hs[i].wait()


def _sc_call(uid, rid, ing, nut, user_t, recipe_t, ingredient_t, nutrition_t):
    def body(uh, rh, ih, nh, ut, rt, it, nt, out_hbm, idx_v, b0, b1, b2, b3,
             g0, g1, g2, g3, s0, s1, s2, s3):
        _body(uh, rh, ih, nh, ut, rt, it, nt, out_hbm, idx_v,
              (b0, b1, b2, b3), (g0, g1, g2, g3), (s0, s1, s2, s3))

    f = pl.kernel(
        body,
        out_type=jax.ShapeDtypeStruct((_BATCH, _NTAB * _DIM), jnp.float32),
        mesh=plsc.VectorSubcoreMesh(core_axis_name="c", subcore_axis_name="s"),
        scratch_types=[
            pltpu.VMEM((_NTAB, _BPW), jnp.int32),
        ] + [pltpu.VMEM((_CHUNK, _DIM), jnp.float32)] * _NBUF
          + [pltpu.SemaphoreType.DMA] * (2 * _NBUF),
        compiler_params=pltpu.CompilerParams(use_tc_tiling_on_sc=False),
    )
    return f(uid, rid, ing, nut, user_t, recipe_t, ingredient_t, nutrition_t)


def kernel(uid, rid, ing, nut, user_table, recipe_table, ingredient_table,
           nutrition_table):
    return _sc_call(uid.astype(jnp.int32), rid.astype(jnp.int32),
                    ing.astype(jnp.int32), nut.astype(jnp.int32),
                    user_table.astype(jnp.float32),
                    recipe_table.astype(jnp.float32),
                    ingredient_table.astype(jnp.float32),
                    nutrition_table.astype(jnp.float32))
